# baseline (speedup 1.0000x reference)
import jax
import jax.numpy as jnp
from jax.experimental import pallas as pl
from jax.experimental.pallas import tpu as pltpu

N = 32768
B = 16
D_IN = 64
D_COORD = 3
D_EMB = 128
NUM_CLASSES = 55

BLK = 8192
GRID = N // BLK


def _fused_kernel(off_ref, coords_ref, feats_ref, wencT_ref, benc_ref,
                  w_ref, b_ref, out_ref, acc_ref):
    g = pl.program_id(0)

    @pl.when(g == 0)
    def _init():
        acc_ref[...] = jnp.zeros_like(acc_ref)

    wcT = wencT_ref[:, 0:D_COORD]
    wfT = wencT_ref[:, D_COORD:D_COORD + D_IN]
    embT = jnp.dot(wcT, coords_ref[...], preferred_element_type=jnp.float32)
    embT += jnp.dot(wfT, feats_ref[...], preferred_element_type=jnp.float32)
    embT = jnp.maximum(embT + benc_ref[...], 0.0)

    bidx = jax.lax.broadcasted_iota(jnp.int32, (1, B), 1)
    lower = jnp.zeros((1, B), jnp.int32)
    upper = jnp.zeros((1, B), jnp.int32)
    for k in range(B):
        lower = jnp.where(bidx == k, off_ref[k], lower)
        upper = jnp.where(bidx == k, off_ref[k + 1], upper)

    rows = g * BLK + jax.lax.broadcasted_iota(jnp.int32, (BLK, 1), 0)
    selT = ((rows >= lower) & (rows < upper)).astype(jnp.float32)

    acc_ref[...] += jnp.dot(embT, selT, preferred_element_type=jnp.float32)

    @pl.when(g == GRID - 1)
    def _finish():
        counts = (upper - lower).astype(jnp.float32)
        gfT = acc_ref[...] / jnp.maximum(counts, 1.0)
        out_ref[...] = jax.lax.dot_general(
            gfT, w_ref[...], (((0,), (0,)), ((), ())),
            preferred_element_type=jnp.float32) + b_ref[...]


@jax.jit
def kernel(coords, feats, batch_offsets, W_enc, b_enc, W, b):
    return pl.pallas_call(
        _fused_kernel,
        grid=(GRID,),
        in_specs=[
            pl.BlockSpec(memory_space=pltpu.SMEM),
            pl.BlockSpec((D_COORD, BLK), lambda g: (0, g)),
            pl.BlockSpec((D_IN, BLK), lambda g: (0, g)),
            pl.BlockSpec((D_EMB, D_COORD + D_IN), lambda g: (0, 0)),
            pl.BlockSpec((D_EMB, 1), lambda g: (0, 0)),
            pl.BlockSpec((D_EMB, NUM_CLASSES), lambda g: (0, 0)),
            pl.BlockSpec((1, NUM_CLASSES), lambda g: (0, 0)),
        ],
        out_specs=pl.BlockSpec((B, NUM_CLASSES), lambda g: (0, 0)),
        out_shape=jax.ShapeDtypeStruct((B, NUM_CLASSES), jnp.float32),
        scratch_shapes=[pltpu.VMEM((D_EMB, B), jnp.float32)],
        compiler_params=pltpu.CompilerParams(
            dimension_semantics=("arbitrary",)),
    )(batch_offsets, coords.T, feats.T, W_enc.T,
      b_enc.reshape(D_EMB, 1), W, b.reshape(1, NUM_CLASSES))


# speedup vs baseline: 1.1410x; 1.1410x over previous
import jax
import jax.numpy as jnp
from jax.experimental import pallas as pl
from jax.experimental.pallas import tpu as pltpu

N = 32768
B = 16
D_IN = 64
D_COORD = 3
D_EMB = 128
NUM_CLASSES = 55

BLK = 8192
GRID = N // BLK


def _fused_kernel(off_ref, coords_ref, feats_ref, wencT_ref, benc_ref,
                  w_ref, b_ref, out_ref, acc_ref):
    g = pl.program_id(0)

    @pl.when(g == 0)
    def _init():
        acc_ref[...] = jnp.zeros_like(acc_ref)

    wcT = wencT_ref[:, 0:D_COORD]
    wfT = wencT_ref[:, D_COORD:D_COORD + D_IN]
    embT = jnp.dot(wcT, coords_ref[...], preferred_element_type=jnp.float32)
    embT += jnp.dot(wfT, feats_ref[...], preferred_element_type=jnp.float32)
    embT = jnp.maximum(embT + benc_ref[...], 0.0)

    bidx = jax.lax.broadcasted_iota(jnp.int32, (1, B), 1)
    lower = jnp.zeros((1, B), jnp.int32)
    upper = jnp.zeros((1, B), jnp.int32)
    for k in range(B):
        lower = jnp.where(bidx == k, off_ref[k], lower)
        upper = jnp.where(bidx == k, off_ref[k + 1], upper)

    bcol = jax.lax.broadcasted_iota(jnp.int32, (B, 1), 0)
    low_c = jnp.zeros((B, 1), jnp.int32)
    up_c = jnp.zeros((B, 1), jnp.int32)
    for k in range(B):
        low_c = jnp.where(bcol == k, off_ref[k], low_c)
        up_c = jnp.where(bcol == k, off_ref[k + 1], up_c)
    rows = g * BLK + jax.lax.broadcasted_iota(jnp.int32, (1, BLK), 1)
    sel = ((rows >= low_c) & (rows < up_c)).astype(jnp.float32)
    selT = jnp.transpose(sel)

    acc_ref[...] += jnp.dot(embT, selT, preferred_element_type=jnp.float32)

    @pl.when(g == GRID - 1)
    def _finish():
        counts = (upper - lower).astype(jnp.float32)
        gfT = acc_ref[...] / jnp.maximum(counts, 1.0)
        out_ref[...] = jax.lax.dot_general(
            gfT, w_ref[...], (((0,), (0,)), ((), ())),
            preferred_element_type=jnp.float32) + b_ref[...]


@jax.jit
def kernel(coords, feats, batch_offsets, W_enc, b_enc, W, b):
    return pl.pallas_call(
        _fused_kernel,
        grid=(GRID,),
        in_specs=[
            pl.BlockSpec(memory_space=pltpu.SMEM),
            pl.BlockSpec((D_COORD, BLK), lambda g: (0, g)),
            pl.BlockSpec((D_IN, BLK), lambda g: (0, g)),
            pl.BlockSpec((D_EMB, D_COORD + D_IN), lambda g: (0, 0)),
            pl.BlockSpec((D_EMB, 1), lambda g: (0, 0)),
            pl.BlockSpec((D_EMB, NUM_CLASSES), lambda g: (0, 0)),
            pl.BlockSpec((1, NUM_CLASSES), lambda g: (0, 0)),
        ],
        out_specs=pl.BlockSpec((B, NUM_CLASSES), lambda g: (0, 0)),
        out_shape=jax.ShapeDtypeStruct((B, NUM_CLASSES), jnp.float32),
        scratch_shapes=[pltpu.VMEM((D_EMB, B), jnp.float32)],
        compiler_params=pltpu.CompilerParams(
            dimension_semantics=("arbitrary",)),
    )(batch_offsets, coords.T, feats.T, W_enc.T,
      b_enc.reshape(D_EMB, 1), W, b.reshape(1, NUM_CLASSES))


# speedup vs baseline: 1.1413x; 1.0002x over previous
import jax
import jax.numpy as jnp
from jax.experimental import pallas as pl
from jax.experimental.pallas import tpu as pltpu

N = 32768
B = 16
D_IN = 64
D_COORD = 3
D_EMB = 128
NUM_CLASSES = 55

BLK = 8192
GRID = N // BLK


def _fused_kernel(off_ref, coords_ref, feats_ref, wencT_ref, benc_ref,
                  w_ref, b_ref, out_ref, acc_ref):
    g = pl.program_id(0)

    @pl.when(g == 0)
    def _init():
        acc_ref[...] = jnp.zeros_like(acc_ref)

    wcT = wencT_ref[:, 0:D_COORD]
    wfT = wencT_ref[:, D_COORD:D_COORD + D_IN]
    cb = coords_ref[...].astype(jnp.bfloat16)
    fb = feats_ref[...].astype(jnp.bfloat16)
    embT = jnp.dot(wcT, cb, preferred_element_type=jnp.float32)
    embT += jnp.dot(wfT, fb, preferred_element_type=jnp.float32)
    embT = jnp.maximum(embT + benc_ref[...], 0.0).astype(jnp.bfloat16)

    bidx = jax.lax.broadcasted_iota(jnp.int32, (1, B), 1)
    lower = jnp.zeros((1, B), jnp.int32)
    upper = jnp.zeros((1, B), jnp.int32)
    for k in range(B):
        lower = jnp.where(bidx == k, off_ref[k], lower)
        upper = jnp.where(bidx == k, off_ref[k + 1], upper)

    bcol = jax.lax.broadcasted_iota(jnp.int32, (B, 1), 0)
    low_c = jnp.zeros((B, 1), jnp.int32)
    up_c = jnp.zeros((B, 1), jnp.int32)
    for k in range(B):
        low_c = jnp.where(bcol == k, off_ref[k], low_c)
        up_c = jnp.where(bcol == k, off_ref[k + 1], up_c)
    rows = g * BLK + jax.lax.broadcasted_iota(jnp.int32, (1, BLK), 1)
    sel = ((rows >= low_c) & (rows < up_c)).astype(jnp.bfloat16)
    selT = jnp.transpose(sel)

    acc_ref[...] += jnp.dot(embT, selT, preferred_element_type=jnp.float32)

    @pl.when(g == GRID - 1)
    def _finish():
        counts = (upper - lower).astype(jnp.float32)
        gfT = acc_ref[...] / jnp.maximum(counts, 1.0)
        out_ref[...] = jax.lax.dot_general(
            gfT, w_ref[...], (((0,), (0,)), ((), ())),
            preferred_element_type=jnp.float32) + b_ref[...]


@jax.jit
def kernel(coords, feats, batch_offsets, W_enc, b_enc, W, b):
    return pl.pallas_call(
        _fused_kernel,
        grid=(GRID,),
        in_specs=[
            pl.BlockSpec(memory_space=pltpu.SMEM),
            pl.BlockSpec((D_COORD, BLK), lambda g: (0, g)),
            pl.BlockSpec((D_IN, BLK), lambda g: (0, g)),
            pl.BlockSpec((D_EMB, D_COORD + D_IN), lambda g: (0, 0)),
            pl.BlockSpec((D_EMB, 1), lambda g: (0, 0)),
            pl.BlockSpec((D_EMB, NUM_CLASSES), lambda g: (0, 0)),
            pl.BlockSpec((1, NUM_CLASSES), lambda g: (0, 0)),
        ],
        out_specs=pl.BlockSpec((B, NUM_CLASSES), lambda g: (0, 0)),
        out_shape=jax.ShapeDtypeStruct((B, NUM_CLASSES), jnp.float32),
        scratch_shapes=[pltpu.VMEM((D_EMB, B), jnp.float32)],
        compiler_params=pltpu.CompilerParams(
            dimension_semantics=("arbitrary",)),
    )(batch_offsets, coords.T, feats.T, W_enc.T.astype(jnp.bfloat16),
      b_enc.reshape(D_EMB, 1), W, b.reshape(1, NUM_CLASSES))
